# baseline (speedup 1.0000x reference)
import functools

import jax
import jax.numpy as jnp
from jax import lax
from jax.experimental import pallas as pl
from jax.experimental.pallas import tpu as pltpu
from jax.experimental.pallas import tpu_sc as plsc

RESO = 128
NPTS = 2_000_000
BLK = 2000
NBLK = NPTS // BLK
NWORKERS = 32
CHUNKS = BLK // 16

_F32 = jnp.float32
_I32 = jnp.int32


def _rsqrt(s):
    i = lax.bitcast_convert_type(s, _I32)
    i = jnp.int32(0x5F3759DF) - lax.shift_right_arithmetic(i, 1)
    y = lax.bitcast_convert_type(i, _F32)
    for _ in range(3):
        y = y * (1.5 - 0.5 * s * y * y)
    return y


def _body(rpts_h, ridx_h, spts_h, sidx_h, xl_h, yl_h, zl_h, off_h,
          sdf_out_h, nrm_out_h,
          xl_v, yl_v, zl_v, off_v, idx_v, pts_v, sdf_v, nrm_v):
    w = lax.axis_index("s") * 2 + lax.axis_index("c")
    pltpu.sync_copy(xl_h, xl_v)
    pltpu.sync_copy(yl_h, yl_v)
    pltpu.sync_copy(zl_h, zl_v)
    pltpu.sync_copy(off_h, off_v)

    iota = lax.broadcasted_iota(_I32, (16,), 0)
    zero = iota * 0
    one = zero + 1
    two = zero + 2
    d = off_v[0]
    e = off_v[1]
    f = off_v[2]
    g = off_v[3]

    nblk_w = (NBLK - 1 - w) // NWORKERS + 1

    def gather_chunk(i):
        s = i * 16
        idx = idx_v[pl.ds(s, 16)]
        iz = lax.bitwise_and(idx, 127)
        iy = lax.bitwise_and(lax.shift_right_logical(idx, 7), 127)
        ix = lax.bitwise_and(lax.shift_right_logical(idx, 14), 127)
        a = plsc.load_gather(xl_v, [ix])
        b = plsc.load_gather(yl_v, [iy])
        c = plsc.load_gather(zl_v, [iz])
        j = iota + s
        px = plsc.load_gather(pts_v, [j, zero]) + ix.astype(_F32)
        py = plsc.load_gather(pts_v, [j, one]) + iy.astype(_F32)
        pz = plsc.load_gather(pts_v, [j, two]) + iz.astype(_F32)
        return a, b, c, px, py, pz, j

    def sdf_block(t, carry):
        base = (w + t * NWORKERS) * BLK
        pltpu.sync_copy(sidx_h.at[pl.ds(base, BLK)], idx_v)
        pltpu.sync_copy(spts_h.at[pl.ds(base, BLK), :], pts_v)

        def chunk(i, c2):
            a, b, c, px, py, pz, _ = gather_chunk(i)
            val = px * (a * px + d) + py * (b * py + e) + pz * (c * pz + f) + g
            sdf_v[pl.ds(i * 16, 16)] = val
            return c2

        lax.fori_loop(0, CHUNKS, chunk, 0)
        pltpu.sync_copy(sdf_v, sdf_out_h.at[pl.ds(base, BLK)])
        return carry

    def nrm_block(t, carry):
        base = (w + t * NWORKERS) * BLK
        pltpu.sync_copy(ridx_h.at[pl.ds(base, BLK)], idx_v)
        pltpu.sync_copy(rpts_h.at[pl.ds(base, BLK), :], pts_v)

        def chunk(i, c2):
            a, b, c, px, py, pz, j = gather_chunk(i)
            gx = 2.0 * a * px + d
            gy = 2.0 * b * py + e
            gz = 2.0 * c * pz + f
            s2 = gx * gx + gy * gy + gz * gz
            norm = s2 * _rsqrt(s2)
            inv = 1.0 / (norm + 1e-8)
            plsc.store_scatter(nrm_v, [j, zero], gx * inv)
            plsc.store_scatter(nrm_v, [j, one], gy * inv)
            plsc.store_scatter(nrm_v, [j, two], gz * inv)
            return c2

        lax.fori_loop(0, CHUNKS, chunk, 0)
        pltpu.sync_copy(nrm_v, nrm_out_h.at[pl.ds(base, BLK), :])
        return carry

    lax.fori_loop(0, nblk_w, sdf_block, 0)
    lax.fori_loop(0, nblk_w, nrm_block, 0)


_sc_call = functools.partial(
    pl.kernel,
    out_type=[
        jax.ShapeDtypeStruct((NPTS,), _F32),
        jax.ShapeDtypeStruct((NPTS, 3), _F32),
    ],
    mesh=plsc.VectorSubcoreMesh(core_axis_name="c", subcore_axis_name="s"),
    compiler_params=pltpu.CompilerParams(
        needs_layout_passes=False, use_tc_tiling_on_sc=False),
    scratch_types=[
        pltpu.VMEM((RESO,), _F32),
        pltpu.VMEM((RESO,), _F32),
        pltpu.VMEM((RESO,), _F32),
        pltpu.VMEM((4, 16), _F32),
        pltpu.VMEM((BLK,), _I32),
        pltpu.VMEM((BLK, 3), _F32),
        pltpu.VMEM((BLK,), _F32),
        pltpu.VMEM((BLK, 3), _F32),
    ],
)(_body)


def kernel(renderPointList, renderIndexList, sdfPointList, sdfIndexList,
           xLayer, yLayer, zLayer, offset):
    off16 = jnp.broadcast_to(offset[:, None], (4, 16))
    sdf, nrm = _sc_call(
        renderPointList, renderIndexList,
        sdfPointList, sdfIndexList,
        xLayer, yLayer, zLayer, off16)
    return (sdf, nrm)


# speedup vs baseline: 7.7708x; 7.7708x over previous
import functools

import jax
import jax.numpy as jnp
from jax import lax
from jax.experimental import pallas as pl
from jax.experimental.pallas import tpu as pltpu
from jax.experimental.pallas import tpu_sc as plsc

RESO = 128
NPTS = 2_000_000
BLK = 2000
NBLK = NPTS // BLK
NWORKERS = 32
CHUNKS = BLK // 16

_F32 = jnp.float32
_I32 = jnp.int32


def _rsqrt(s):
    i = lax.bitcast_convert_type(s, _I32)
    i = jnp.int32(0x5F3759DF) - lax.shift_right_arithmetic(i, 1)
    y = lax.bitcast_convert_type(i, _F32)
    for _ in range(3):
        y = y * (1.5 - 0.5 * s * y * y)
    return y


def _body(rpts_h, ridx_h, spts_h, sidx_h, xl_h, yl_h, zl_h, off_h,
          sdf_out_h, nrm_out_h,
          xl_v, yl_v, zl_v, off_v, idx_v, px_v, py_v, pz_v,
          ox_v, oy_v, oz_v):
    w = lax.axis_index("s") * 2 + lax.axis_index("c")
    pltpu.sync_copy(xl_h, xl_v)
    pltpu.sync_copy(yl_h, yl_v)
    pltpu.sync_copy(zl_h, zl_v)
    pltpu.sync_copy(off_h, off_v)

    d = off_v[0]
    e = off_v[1]
    f = off_v[2]
    g = off_v[3]

    nblk_w = (NBLK - 1 - w) // NWORKERS + 1

    def gather_chunk(i):
        s = i * 16
        idx = idx_v[pl.ds(s, 16)]
        iz = lax.bitwise_and(idx, 127)
        iy = lax.bitwise_and(lax.shift_right_logical(idx, 7), 127)
        ix = lax.bitwise_and(lax.shift_right_logical(idx, 14), 127)
        a = plsc.load_gather(xl_v, [ix])
        b = plsc.load_gather(yl_v, [iy])
        c = plsc.load_gather(zl_v, [iz])
        px = px_v[pl.ds(s, 16)] + ix.astype(_F32)
        py = py_v[pl.ds(s, 16)] + iy.astype(_F32)
        pz = pz_v[pl.ds(s, 16)] + iz.astype(_F32)
        return a, b, c, px, py, pz

    def load_block(pts_h, idx_h, base):
        pltpu.sync_copy(idx_h.at[pl.ds(base, BLK)], idx_v)
        pltpu.sync_copy(pts_h.at[0, pl.ds(base, BLK)], px_v)
        pltpu.sync_copy(pts_h.at[1, pl.ds(base, BLK)], py_v)
        pltpu.sync_copy(pts_h.at[2, pl.ds(base, BLK)], pz_v)

    def sdf_block(t, carry):
        base = (w + t * NWORKERS) * BLK
        load_block(spts_h, sidx_h, base)

        def chunk(i, c2):
            a, b, c, px, py, pz = gather_chunk(i)
            val = px * (a * px + d) + py * (b * py + e) + pz * (c * pz + f) + g
            ox_v[pl.ds(i * 16, 16)] = val
            return c2

        lax.fori_loop(0, CHUNKS, chunk, 0)
        pltpu.sync_copy(ox_v, sdf_out_h.at[pl.ds(base, BLK)])
        return carry

    def nrm_block(t, carry):
        base = (w + t * NWORKERS) * BLK
        load_block(rpts_h, ridx_h, base)

        def chunk(i, c2):
            a, b, c, px, py, pz = gather_chunk(i)
            gx = 2.0 * a * px + d
            gy = 2.0 * b * py + e
            gz = 2.0 * c * pz + f
            s2 = gx * gx + gy * gy + gz * gz
            norm = s2 * _rsqrt(s2)
            inv = 1.0 / (norm + 1e-8)
            s = i * 16
            ox_v[pl.ds(s, 16)] = gx * inv
            oy_v[pl.ds(s, 16)] = gy * inv
            oz_v[pl.ds(s, 16)] = gz * inv
            return c2

        lax.fori_loop(0, CHUNKS, chunk, 0)
        pltpu.sync_copy(ox_v, nrm_out_h.at[0, pl.ds(base, BLK)])
        pltpu.sync_copy(oy_v, nrm_out_h.at[1, pl.ds(base, BLK)])
        pltpu.sync_copy(oz_v, nrm_out_h.at[2, pl.ds(base, BLK)])
        return carry

    lax.fori_loop(0, nblk_w, sdf_block, 0)
    lax.fori_loop(0, nblk_w, nrm_block, 0)


_sc_call = functools.partial(
    pl.kernel,
    out_type=[
        jax.ShapeDtypeStruct((NPTS,), _F32),
        jax.ShapeDtypeStruct((3, NPTS), _F32),
    ],
    mesh=plsc.VectorSubcoreMesh(core_axis_name="c", subcore_axis_name="s"),
    compiler_params=pltpu.CompilerParams(
        needs_layout_passes=False, use_tc_tiling_on_sc=False),
    scratch_types=[
        pltpu.VMEM((RESO,), _F32),
        pltpu.VMEM((RESO,), _F32),
        pltpu.VMEM((RESO,), _F32),
        pltpu.VMEM((4, 16), _F32),
        pltpu.VMEM((BLK,), _I32),
        pltpu.VMEM((BLK,), _F32),
        pltpu.VMEM((BLK,), _F32),
        pltpu.VMEM((BLK,), _F32),
        pltpu.VMEM((BLK,), _F32),
        pltpu.VMEM((BLK,), _F32),
        pltpu.VMEM((BLK,), _F32),
    ],
)(_body)


def kernel(renderPointList, renderIndexList, sdfPointList, sdfIndexList,
           xLayer, yLayer, zLayer, offset):
    off16 = jnp.broadcast_to(offset[:, None], (4, 16))
    sdf, nrm = _sc_call(
        renderPointList.T, renderIndexList,
        sdfPointList.T, sdfIndexList,
        xLayer, yLayer, zLayer, off16)
    return (sdf, nrm.T)


# speedup vs baseline: 20.3824x; 2.6230x over previous
import functools

import jax
import jax.numpy as jnp
from jax import lax
from jax.experimental import pallas as pl
from jax.experimental.pallas import tpu as pltpu
from jax.experimental.pallas import tpu_sc as plsc

RESO = 128
NPTS = 2_000_000
BLK = 2000
NBLK = NPTS // BLK
NWORKERS = 32
CHUNKS = BLK // 16

_F32 = jnp.float32
_I32 = jnp.int32


def _rsqrt(s):
    i = lax.bitcast_convert_type(s, _I32)
    i = jnp.int32(0x5F3759DF) - lax.shift_right_arithmetic(i, 1)
    y = lax.bitcast_convert_type(i, _F32)
    for _ in range(3):
        y = y * (1.5 - 0.5 * s * y * y)
    return y


def _body(rx_h, ry_h, rz_h, ridx_h, sx_h, sy_h, sz_h, sidx_h,
          xl_h, yl_h, zl_h, off_h,
          sdf_out_h, nx_out_h, ny_out_h, nz_out_h,
          xl_v, yl_v, zl_v, off_v, idx_v, px_v, py_v, pz_v,
          ox_v, oy_v, oz_v):
    w = lax.axis_index("s") * 2 + lax.axis_index("c")
    pltpu.sync_copy(xl_h, xl_v)
    pltpu.sync_copy(yl_h, yl_v)
    pltpu.sync_copy(zl_h, zl_v)
    pltpu.sync_copy(off_h, off_v)

    d = off_v[0]
    e = off_v[1]
    f = off_v[2]
    g = off_v[3]

    nblk_w = (NBLK - 1 - w) // NWORKERS + 1

    def gather_chunk(i):
        s = i * 16
        idx = idx_v[pl.ds(s, 16)]
        iz = lax.bitwise_and(idx, 127)
        iy = lax.bitwise_and(lax.shift_right_logical(idx, 7), 127)
        ix = lax.bitwise_and(lax.shift_right_logical(idx, 14), 127)
        a = plsc.load_gather(xl_v, [ix])
        b = plsc.load_gather(yl_v, [iy])
        c = plsc.load_gather(zl_v, [iz])
        px = px_v[pl.ds(s, 16)] + ix.astype(_F32)
        py = py_v[pl.ds(s, 16)] + iy.astype(_F32)
        pz = pz_v[pl.ds(s, 16)] + iz.astype(_F32)
        return a, b, c, px, py, pz

    def load_block(x_h, y_h, z_h, idx_h, base):
        pltpu.sync_copy(idx_h.at[pl.ds(base, BLK)], idx_v)
        pltpu.sync_copy(x_h.at[pl.ds(base, BLK)], px_v)
        pltpu.sync_copy(y_h.at[pl.ds(base, BLK)], py_v)
        pltpu.sync_copy(z_h.at[pl.ds(base, BLK)], pz_v)

    def sdf_block(t, carry):
        base = (w + t * NWORKERS) * BLK
        load_block(sx_h, sy_h, sz_h, sidx_h, base)

        def chunk(i, c2):
            a, b, c, px, py, pz = gather_chunk(i)
            val = px * (a * px + d) + py * (b * py + e) + pz * (c * pz + f) + g
            ox_v[pl.ds(i * 16, 16)] = val
            return c2

        lax.fori_loop(0, CHUNKS, chunk, 0)
        pltpu.sync_copy(ox_v, sdf_out_h.at[pl.ds(base, BLK)])
        return carry

    def nrm_block(t, carry):
        base = (w + t * NWORKERS) * BLK
        load_block(rx_h, ry_h, rz_h, ridx_h, base)

        def chunk(i, c2):
            a, b, c, px, py, pz = gather_chunk(i)
            gx = 2.0 * a * px + d
            gy = 2.0 * b * py + e
            gz = 2.0 * c * pz + f
            s2 = gx * gx + gy * gy + gz * gz
            norm = s2 * _rsqrt(s2)
            inv = 1.0 / (norm + 1e-8)
            s = i * 16
            ox_v[pl.ds(s, 16)] = gx * inv
            oy_v[pl.ds(s, 16)] = gy * inv
            oz_v[pl.ds(s, 16)] = gz * inv
            return c2

        lax.fori_loop(0, CHUNKS, chunk, 0)
        pltpu.sync_copy(ox_v, nx_out_h.at[pl.ds(base, BLK)])
        pltpu.sync_copy(oy_v, ny_out_h.at[pl.ds(base, BLK)])
        pltpu.sync_copy(oz_v, nz_out_h.at[pl.ds(base, BLK)])
        return carry

    lax.fori_loop(0, nblk_w, sdf_block, 0)
    lax.fori_loop(0, nblk_w, nrm_block, 0)


_sc_call = functools.partial(
    pl.kernel,
    out_type=[
        jax.ShapeDtypeStruct((NPTS,), _F32),
        jax.ShapeDtypeStruct((NPTS,), _F32),
        jax.ShapeDtypeStruct((NPTS,), _F32),
        jax.ShapeDtypeStruct((NPTS,), _F32),
    ],
    mesh=plsc.VectorSubcoreMesh(core_axis_name="c", subcore_axis_name="s"),
    compiler_params=pltpu.CompilerParams(
        needs_layout_passes=False, use_tc_tiling_on_sc=False),
    scratch_types=[
        pltpu.VMEM((RESO,), _F32),
        pltpu.VMEM((RESO,), _F32),
        pltpu.VMEM((RESO,), _F32),
        pltpu.VMEM((4, 16), _F32),
        pltpu.VMEM((BLK,), _I32),
        pltpu.VMEM((BLK,), _F32),
        pltpu.VMEM((BLK,), _F32),
        pltpu.VMEM((BLK,), _F32),
        pltpu.VMEM((BLK,), _F32),
        pltpu.VMEM((BLK,), _F32),
        pltpu.VMEM((BLK,), _F32),
    ],
)(_body)


def kernel(renderPointList, renderIndexList, sdfPointList, sdfIndexList,
           xLayer, yLayer, zLayer, offset):
    off16 = jnp.broadcast_to(offset[:, None], (4, 16))
    sdf, nx, ny, nz = _sc_call(
        renderPointList[:, 0], renderPointList[:, 1], renderPointList[:, 2],
        renderIndexList,
        sdfPointList[:, 0], sdfPointList[:, 1], sdfPointList[:, 2],
        sdfIndexList,
        xLayer, yLayer, zLayer, off16)
    return (sdf, jnp.stack([nx, ny, nz], axis=-1))


# speedup vs baseline: 25.3010x; 1.2413x over previous
import functools

import jax
import jax.numpy as jnp
from jax import lax
from jax.experimental import pallas as pl
from jax.experimental.pallas import tpu as pltpu
from jax.experimental.pallas import tpu_sc as plsc

RESO = 128
NPTS = 2_000_000
BLK = 2000
NBLK = NPTS // BLK
NWORKERS = 32
CHUNKS = BLK // 16

_F32 = jnp.float32
_I32 = jnp.int32


def _rsqrt(s):
    i = lax.bitcast_convert_type(s, _I32)
    i = jnp.int32(0x5F3759DF) - lax.shift_right_arithmetic(i, 1)
    y = lax.bitcast_convert_type(i, _F32)
    for _ in range(3):
        y = y * (1.5 - 0.5 * s * y * y)
    return y


def _body(rx_h, ry_h, rz_h, ridx_h, sx_h, sy_h, sz_h, sidx_h,
          xl_h, yl_h, zl_h, off_h,
          sdf_out_h, nx_out_h, ny_out_h, nz_out_h,
          xl_v, yl_v, zl_v, off_v, idx_v, px_v, py_v, pz_v,
          ox_v, oy_v, oz_v):
    w = lax.axis_index("s") * 2 + lax.axis_index("c")
    pltpu.sync_copy(xl_h, xl_v)
    pltpu.sync_copy(yl_h, yl_v)
    pltpu.sync_copy(zl_h, zl_v)
    pltpu.sync_copy(off_h, off_v)

    d = off_v[0]
    e = off_v[1]
    f = off_v[2]
    g = off_v[3]

    nblk_w = (NBLK - 1 - w) // NWORKERS + 1

    def gather_chunk(i):
        s = i * 16
        idx = idx_v[pl.ds(s, 16)]
        iz = lax.bitwise_and(idx, 127)
        iy = lax.bitwise_and(lax.shift_right_logical(idx, 7), 127)
        ix = lax.bitwise_and(lax.shift_right_logical(idx, 14), 127)
        a = plsc.load_gather(xl_v, [ix])
        b = plsc.load_gather(yl_v, [iy])
        c = plsc.load_gather(zl_v, [iz])
        px = px_v[pl.ds(s, 16)] + ix.astype(_F32)
        py = py_v[pl.ds(s, 16)] + iy.astype(_F32)
        pz = pz_v[pl.ds(s, 16)] + iz.astype(_F32)
        return a, b, c, px, py, pz

    def load_block(x_h, y_h, z_h, idx_h, base):
        pltpu.sync_copy(idx_h.at[pl.ds(base, BLK)], idx_v)
        pltpu.sync_copy(x_h.at[pl.ds(base, BLK)], px_v)
        pltpu.sync_copy(y_h.at[pl.ds(base, BLK)], py_v)
        pltpu.sync_copy(z_h.at[pl.ds(base, BLK)], pz_v)

    def sdf_block(t, carry):
        base = (w + t * NWORKERS) * BLK
        load_block(sx_h, sy_h, sz_h, sidx_h, base)

        @plsc.parallel_loop(0, CHUNKS, 1, unroll=4)
        def _(i):
            a, b, c, px, py, pz = gather_chunk(i)
            val = px * (a * px + d) + py * (b * py + e) + pz * (c * pz + f) + g
            ox_v[pl.ds(i * 16, 16)] = val
        pltpu.sync_copy(ox_v, sdf_out_h.at[pl.ds(base, BLK)])
        return carry

    def nrm_block(t, carry):
        base = (w + t * NWORKERS) * BLK
        load_block(rx_h, ry_h, rz_h, ridx_h, base)

        @plsc.parallel_loop(0, CHUNKS, 1, unroll=4)
        def _(i):
            a, b, c, px, py, pz = gather_chunk(i)
            gx = 2.0 * a * px + d
            gy = 2.0 * b * py + e
            gz = 2.0 * c * pz + f
            s2 = gx * gx + gy * gy + gz * gz
            norm = s2 * _rsqrt(s2)
            inv = 1.0 / (norm + 1e-8)
            s = i * 16
            ox_v[pl.ds(s, 16)] = gx * inv
            oy_v[pl.ds(s, 16)] = gy * inv
            oz_v[pl.ds(s, 16)] = gz * inv
        pltpu.sync_copy(ox_v, nx_out_h.at[pl.ds(base, BLK)])
        pltpu.sync_copy(oy_v, ny_out_h.at[pl.ds(base, BLK)])
        pltpu.sync_copy(oz_v, nz_out_h.at[pl.ds(base, BLK)])
        return carry

    lax.fori_loop(0, nblk_w, sdf_block, 0)
    lax.fori_loop(0, nblk_w, nrm_block, 0)


_sc_call = functools.partial(
    pl.kernel,
    out_type=[
        jax.ShapeDtypeStruct((NPTS,), _F32),
        jax.ShapeDtypeStruct((NPTS,), _F32),
        jax.ShapeDtypeStruct((NPTS,), _F32),
        jax.ShapeDtypeStruct((NPTS,), _F32),
    ],
    mesh=plsc.VectorSubcoreMesh(core_axis_name="c", subcore_axis_name="s"),
    compiler_params=pltpu.CompilerParams(
        needs_layout_passes=False, use_tc_tiling_on_sc=False),
    scratch_types=[
        pltpu.VMEM((RESO,), _F32),
        pltpu.VMEM((RESO,), _F32),
        pltpu.VMEM((RESO,), _F32),
        pltpu.VMEM((4, 16), _F32),
        pltpu.VMEM((BLK,), _I32),
        pltpu.VMEM((BLK,), _F32),
        pltpu.VMEM((BLK,), _F32),
        pltpu.VMEM((BLK,), _F32),
        pltpu.VMEM((BLK,), _F32),
        pltpu.VMEM((BLK,), _F32),
        pltpu.VMEM((BLK,), _F32),
    ],
)(_body)


def kernel(renderPointList, renderIndexList, sdfPointList, sdfIndexList,
           xLayer, yLayer, zLayer, offset):
    off16 = jnp.broadcast_to(offset[:, None], (4, 16))
    sdf, nx, ny, nz = _sc_call(
        renderPointList[:, 0], renderPointList[:, 1], renderPointList[:, 2],
        renderIndexList,
        sdfPointList[:, 0], sdfPointList[:, 1], sdfPointList[:, 2],
        sdfIndexList,
        xLayer, yLayer, zLayer, off16)
    return (sdf, jnp.stack([nx, ny, nz], axis=-1))


# speedup vs baseline: 36.6420x; 1.4482x over previous
import functools

import jax
import jax.numpy as jnp
from jax import lax
from jax.experimental import pallas as pl
from jax.experimental.pallas import tpu as pltpu
from jax.experimental.pallas import tpu_sc as plsc

RESO = 128
NPTS = 2_000_000
BLK = 4000
NBLK = NPTS // BLK
NWORKERS = 32
NT = (NBLK + NWORKERS - 1) // NWORKERS
CHUNKS = BLK // 16

_F32 = jnp.float32
_I32 = jnp.int32


def _rsqrt(s):
    i = lax.bitcast_convert_type(s, _I32)
    i = jnp.int32(0x5F3759DF) - lax.shift_right_arithmetic(i, 1)
    y = lax.bitcast_convert_type(i, _F32)
    for _ in range(3):
        y = y * (1.5 - 0.5 * s * y * y)
    return y


def _body(rx_h, ry_h, rz_h, ridx_h, sx_h, sy_h, sz_h, sidx_h,
          xl_h, yl_h, zl_h, off_h,
          sdf_out_h, nx_out_h, ny_out_h, nz_out_h,
          xl_v, yl_v, zl_v, off_v, idx_b, px_b, py_b, pz_b,
          ox_b, oy_b, oz_b, isem0, isem1, osem0, osem1):
    w = lax.axis_index("s") * 2 + lax.axis_index("c")
    pltpu.sync_copy(xl_h, xl_v)
    pltpu.sync_copy(yl_h, yl_v)
    pltpu.sync_copy(zl_h, zl_v)
    pltpu.sync_copy(off_h, off_v)

    isem = (isem0, isem1)
    osem = (osem0, osem1)
    d = off_v[0]
    e = off_v[1]
    f = off_v[2]
    g = off_v[3]

    nblk_w = (NBLK - 1 - w) // NWORKERS + 1

    def run_phase(x_h, y_h, z_h, idx_h, n_out, out_hs, compute):
        def hbase(k):
            return (w + k * NWORKERS) * BLK

        def in_start(k, sl):
            base = hbase(k)
            sem = isem[sl]
            pltpu.async_copy(idx_h.at[pl.ds(base, BLK)], idx_b.at[sl], sem)
            pltpu.async_copy(x_h.at[pl.ds(base, BLK)], px_b.at[sl], sem)
            pltpu.async_copy(y_h.at[pl.ds(base, BLK)], py_b.at[sl], sem)
            pltpu.async_copy(z_h.at[pl.ds(base, BLK)], pz_b.at[sl], sem)

        def in_wait(sl):
            sem = isem[sl]
            pltpu.make_async_copy(idx_h.at[pl.ds(0, BLK)], idx_b.at[sl], sem).wait()
            pltpu.make_async_copy(x_h.at[pl.ds(0, BLK)], px_b.at[sl], sem).wait()
            pltpu.make_async_copy(y_h.at[pl.ds(0, BLK)], py_b.at[sl], sem).wait()
            pltpu.make_async_copy(z_h.at[pl.ds(0, BLK)], pz_b.at[sl], sem).wait()

        out_bufs = (ox_b, oy_b, oz_b)[:n_out]

        def out_start(k, sl):
            base = hbase(k)
            for buf, hb in zip(out_bufs, out_hs):
                pltpu.async_copy(buf.at[sl], hb.at[pl.ds(base, BLK)], osem[sl])

        def out_wait(sl):
            for buf, hb in zip(out_bufs, out_hs):
                pltpu.make_async_copy(buf.at[sl], hb.at[pl.ds(0, BLK)], osem[sl]).wait()

        @pl.when(nblk_w > 0)
        def _():
            in_start(0, 0)

        def pair(k2, carry):
            for b2 in (0, 1):
                k = 2 * k2 + b2

                @pl.when(k + 1 < nblk_w)
                def _():
                    in_start(k + 1, 1 - b2)

                @pl.when(k < nblk_w)
                def _():
                    in_wait(b2)

                    @pl.when(k >= 2)
                    def _():
                        out_wait(b2)

                    compute(b2)
                    out_start(k, b2)
            return carry

        lax.fori_loop(0, NT // 2, pair, 0)
        out_wait(0)

        @pl.when(nblk_w >= 2)
        def _():
            out_wait(1)

    def decode(sl, i):
        s = i * 16
        idx = idx_b.at[sl][pl.ds(s, 16)]
        iz = lax.bitwise_and(idx, 127)
        iy = lax.bitwise_and(lax.shift_right_logical(idx, 7), 127)
        ix = lax.shift_right_logical(idx, 14)
        a = plsc.load_gather(xl_v, [ix])
        b = plsc.load_gather(yl_v, [iy])
        c = plsc.load_gather(zl_v, [iz])
        px = px_b.at[sl][pl.ds(s, 16)] + ix.astype(_F32)
        py = py_b.at[sl][pl.ds(s, 16)] + iy.astype(_F32)
        pz = pz_b.at[sl][pl.ds(s, 16)] + iz.astype(_F32)
        return a, b, c, px, py, pz

    def sdf_compute(sl):
        @plsc.parallel_loop(0, CHUNKS, 1, unroll=4)
        def _(i):
            a, b, c, px, py, pz = decode(sl, i)
            val = px * (a * px + d) + py * (b * py + e) + pz * (c * pz + f) + g
            ox_b.at[sl][pl.ds(i * 16, 16)] = val

    def nrm_compute(sl):
        @plsc.parallel_loop(0, CHUNKS, 1, unroll=4)
        def _(i):
            a, b, c, px, py, pz = decode(sl, i)
            gx = 2.0 * a * px + d
            gy = 2.0 * b * py + e
            gz = 2.0 * c * pz + f
            s2 = gx * gx + gy * gy + gz * gz
            norm = s2 * _rsqrt(s2)
            inv = 1.0 / (norm + 1e-8)
            s = i * 16
            ox_b.at[sl][pl.ds(s, 16)] = gx * inv
            oy_b.at[sl][pl.ds(s, 16)] = gy * inv
            oz_b.at[sl][pl.ds(s, 16)] = gz * inv

    run_phase(sx_h, sy_h, sz_h, sidx_h, 1, (sdf_out_h,), sdf_compute)
    run_phase(rx_h, ry_h, rz_h, ridx_h, 3, (nx_out_h, ny_out_h, nz_out_h),
              nrm_compute)


_sc_call = functools.partial(
    pl.kernel,
    out_type=[
        jax.ShapeDtypeStruct((NPTS,), _F32),
        jax.ShapeDtypeStruct((NPTS,), _F32),
        jax.ShapeDtypeStruct((NPTS,), _F32),
        jax.ShapeDtypeStruct((NPTS,), _F32),
    ],
    mesh=plsc.VectorSubcoreMesh(core_axis_name="c", subcore_axis_name="s"),
    compiler_params=pltpu.CompilerParams(
        needs_layout_passes=False, use_tc_tiling_on_sc=False),
    scratch_types=[
        pltpu.VMEM((RESO,), _F32),
        pltpu.VMEM((RESO,), _F32),
        pltpu.VMEM((RESO,), _F32),
        pltpu.VMEM((4, 16), _F32),
        pltpu.VMEM((2, BLK), _I32),
        pltpu.VMEM((2, BLK), _F32),
        pltpu.VMEM((2, BLK), _F32),
        pltpu.VMEM((2, BLK), _F32),
        pltpu.VMEM((2, BLK), _F32),
        pltpu.VMEM((2, BLK), _F32),
        pltpu.VMEM((2, BLK), _F32),
        pltpu.SemaphoreType.DMA,
        pltpu.SemaphoreType.DMA,
        pltpu.SemaphoreType.DMA,
        pltpu.SemaphoreType.DMA,
    ],
)(_body)


def kernel(renderPointList, renderIndexList, sdfPointList, sdfIndexList,
           xLayer, yLayer, zLayer, offset):
    off16 = jnp.broadcast_to(offset[:, None], (4, 16))
    sdf, nx, ny, nz = _sc_call(
        renderPointList[:, 0], renderPointList[:, 1], renderPointList[:, 2],
        renderIndexList,
        sdfPointList[:, 0], sdfPointList[:, 1], sdfPointList[:, 2],
        sdfIndexList,
        xLayer, yLayer, zLayer, off16)
    return (sdf, jnp.stack([nx, ny, nz], axis=-1))


# speedup vs baseline: 44.6778x; 1.2193x over previous
import functools

import jax
import jax.numpy as jnp
from jax import lax
from jax.experimental import pallas as pl
from jax.experimental.pallas import tpu as pltpu
from jax.experimental.pallas import tpu_sc as plsc

RESO = 128
NPTS = 2_000_000
BLK = 4000
NBLK = NPTS // BLK
NWORKERS = 32
NT = (NBLK + NWORKERS - 1) // NWORKERS
CHUNKS = BLK // 16

_F32 = jnp.float32
_I32 = jnp.int32


def _rsqrt(s):
    i = lax.bitcast_convert_type(s, _I32)
    i = jnp.int32(0x5F3759DF) - lax.shift_right_arithmetic(i, 1)
    y = lax.bitcast_convert_type(i, _F32)
    for _ in range(3):
        y = y * (1.5 - 0.5 * s * y * y)
    return y


def _make_body(is_sdf):
    n_out = 1 if is_sdf else 3

    def _body(*args):
        (x_h, y_h, z_h, idx_h, xl_h, yl_h, zl_h, off_h), rest = args[:8], args[8:]
        out_hs, rest = rest[:n_out], rest[n_out:]
        (xl_v, yl_v, zl_v, off_v, idx_b, px_b, py_b, pz_b) = rest[:8]
        out_bufs = rest[8:8 + n_out]
        isem0, isem1, osem0, osem1 = rest[8 + n_out:]

        w = lax.axis_index("s") * 2 + lax.axis_index("c")
        pltpu.sync_copy(xl_h, xl_v)
        pltpu.sync_copy(yl_h, yl_v)
        pltpu.sync_copy(zl_h, zl_v)
        pltpu.sync_copy(off_h, off_v)

        isem = (isem0, isem1)
        osem = (osem0, osem1)
        d = off_v[0]
        e = off_v[1]
        f = off_v[2]
        g = off_v[3]

        nblk_w = (NBLK - 1 - w) // NWORKERS + 1

        def hbase(k):
            return (w + k * NWORKERS) * BLK

        def in_start(k, sl):
            base = hbase(k)
            sem = isem[sl]
            pltpu.async_copy(idx_h.at[pl.ds(base, BLK)], idx_b.at[sl], sem)
            pltpu.async_copy(x_h.at[pl.ds(base, BLK)], px_b.at[sl], sem)
            pltpu.async_copy(y_h.at[pl.ds(base, BLK)], py_b.at[sl], sem)
            pltpu.async_copy(z_h.at[pl.ds(base, BLK)], pz_b.at[sl], sem)

        def in_wait(sl):
            sem = isem[sl]
            pltpu.make_async_copy(idx_h.at[pl.ds(0, BLK)], idx_b.at[sl], sem).wait()
            pltpu.make_async_copy(x_h.at[pl.ds(0, BLK)], px_b.at[sl], sem).wait()
            pltpu.make_async_copy(y_h.at[pl.ds(0, BLK)], py_b.at[sl], sem).wait()
            pltpu.make_async_copy(z_h.at[pl.ds(0, BLK)], pz_b.at[sl], sem).wait()

        def out_start(k, sl):
            base = hbase(k)
            for buf, hb in zip(out_bufs, out_hs):
                pltpu.async_copy(buf.at[sl], hb.at[pl.ds(base, BLK)], osem[sl])

        def out_wait(sl):
            for buf, hb in zip(out_bufs, out_hs):
                pltpu.make_async_copy(buf.at[sl], hb.at[pl.ds(0, BLK)], osem[sl]).wait()

        def decode(sl, i):
            s = i * 16
            idx = idx_b.at[sl][pl.ds(s, 16)]
            iz = lax.bitwise_and(idx, 127)
            iy = lax.bitwise_and(lax.shift_right_logical(idx, 7), 127)
            ix = lax.shift_right_logical(idx, 14)
            a = plsc.load_gather(xl_v, [ix])
            b = plsc.load_gather(yl_v, [iy])
            c = plsc.load_gather(zl_v, [iz])
            px = px_b.at[sl][pl.ds(s, 16)] + ix.astype(_F32)
            py = py_b.at[sl][pl.ds(s, 16)] + iy.astype(_F32)
            pz = pz_b.at[sl][pl.ds(s, 16)] + iz.astype(_F32)
            return a, b, c, px, py, pz

        def compute(sl):
            if is_sdf:
                @plsc.parallel_loop(0, CHUNKS, 1, unroll=4)
                def _(i):
                    a, b, c, px, py, pz = decode(sl, i)
                    val = (px * (a * px + d) + py * (b * py + e)
                           + pz * (c * pz + f) + g)
                    out_bufs[0].at[sl][pl.ds(i * 16, 16)] = val
            else:
                @plsc.parallel_loop(0, CHUNKS, 1, unroll=4)
                def _(i):
                    a, b, c, px, py, pz = decode(sl, i)
                    gx = 2.0 * a * px + d
                    gy = 2.0 * b * py + e
                    gz = 2.0 * c * pz + f
                    s2 = gx * gx + gy * gy + gz * gz
                    norm = s2 * _rsqrt(s2)
                    inv = 1.0 / (norm + 1e-8)
                    s = i * 16
                    out_bufs[0].at[sl][pl.ds(s, 16)] = gx * inv
                    out_bufs[1].at[sl][pl.ds(s, 16)] = gy * inv
                    out_bufs[2].at[sl][pl.ds(s, 16)] = gz * inv

        @pl.when(nblk_w > 0)
        def _():
            in_start(0, 0)

        def pair(k2, carry):
            for b2 in (0, 1):
                k = 2 * k2 + b2

                @pl.when(k + 1 < nblk_w)
                def _():
                    in_start(k + 1, 1 - b2)

                @pl.when(k < nblk_w)
                def _():
                    in_wait(b2)

                    @pl.when(k >= 2)
                    def _():
                        out_wait(b2)

                    compute(b2)
                    out_start(k, b2)
            return carry

        lax.fori_loop(0, (NT + 1) // 2, pair, 0)
        out_wait(0)

        @pl.when(nblk_w >= 2)
        def _():
            out_wait(1)

    return _body


def _make_call(is_sdf):
    n_out = 1 if is_sdf else 3
    return functools.partial(
        pl.kernel,
        out_type=[jax.ShapeDtypeStruct((NPTS,), _F32)] * n_out,
        mesh=plsc.VectorSubcoreMesh(core_axis_name="c", subcore_axis_name="s"),
        compiler_params=pltpu.CompilerParams(
            needs_layout_passes=False, use_tc_tiling_on_sc=False),
        scratch_types=[
            pltpu.VMEM((RESO,), _F32),
            pltpu.VMEM((RESO,), _F32),
            pltpu.VMEM((RESO,), _F32),
            pltpu.VMEM((4, 16), _F32),
            pltpu.VMEM((2, BLK), _I32),
            pltpu.VMEM((2, BLK), _F32),
            pltpu.VMEM((2, BLK), _F32),
            pltpu.VMEM((2, BLK), _F32),
        ] + [pltpu.VMEM((2, BLK), _F32)] * n_out
        + [
            pltpu.SemaphoreType.DMA,
            pltpu.SemaphoreType.DMA,
            pltpu.SemaphoreType.DMA,
            pltpu.SemaphoreType.DMA,
        ],
    )(_make_body(is_sdf))


_sdf_call = _make_call(True)
_nrm_call = _make_call(False)


def kernel(renderPointList, renderIndexList, sdfPointList, sdfIndexList,
           xLayer, yLayer, zLayer, offset):
    off16 = jnp.broadcast_to(offset[:, None], (4, 16))
    sdf, = _sdf_call(
        sdfPointList[:, 0], sdfPointList[:, 1], sdfPointList[:, 2],
        sdfIndexList, xLayer, yLayer, zLayer, off16)
    nx, ny, nz = _nrm_call(
        renderPointList[:, 0], renderPointList[:, 1], renderPointList[:, 2],
        renderIndexList, xLayer, yLayer, zLayer, off16)
    return (sdf, jnp.stack([nx, ny, nz], axis=-1))


# speedup vs baseline: 56.3931x; 1.2622x over previous
import functools

import jax
import jax.numpy as jnp
from jax import lax
from jax.experimental import pallas as pl
from jax.experimental.pallas import tpu as pltpu
from jax.experimental.pallas import tpu_sc as plsc

RESO = 128
NPTS = 2_000_000
BLK = 4000
NBLK = NPTS // BLK
NWORKERS = 32
NT = (NBLK + NWORKERS - 1) // NWORKERS
CHUNKS = BLK // 16

_F32 = jnp.float32
_I32 = jnp.int32


def _rsqrt(s):
    i = lax.bitcast_convert_type(s, _I32)
    i = jnp.int32(0x5F3759DF) - lax.shift_right_arithmetic(i, 1)
    y = lax.bitcast_convert_type(i, _F32)
    for _ in range(3):
        y = y * (1.5 - 0.5 * s * y * y)
    return y


def _make_body(is_sdf):
    n_out = 1 if is_sdf else 3

    def _body(*args):
        (x_h, y_h, z_h, idx_h, xl_h, yl_h, zl_h, off_h), rest = args[:8], args[8:]
        out_hs, rest = rest[:n_out], rest[n_out:]
        (xl_v, yl_v, zl_v, off_v, idx_b, px_b, py_b, pz_b) = rest[:8]
        out_bufs = rest[8:8 + n_out]
        isem0, isem1, osem0, osem1 = rest[8 + n_out:]

        w = lax.axis_index("s") * 2 + lax.axis_index("c")
        pltpu.sync_copy(xl_h, xl_v)
        pltpu.sync_copy(yl_h, yl_v)
        pltpu.sync_copy(zl_h, zl_v)
        pltpu.sync_copy(off_h, off_v)

        isem = (isem0, isem1)
        osem = (osem0, osem1)
        d = off_v[0]
        e = off_v[1]
        f = off_v[2]
        g = off_v[3]

        nblk_w = (NBLK - 1 - w) // NWORKERS + 1

        def hbase(k):
            return (w + k * NWORKERS) * BLK

        def in_start(k, sl):
            base = hbase(k)
            sem = isem[sl]
            pltpu.async_copy(idx_h.at[pl.ds(base, BLK)], idx_b.at[sl], sem)
            pltpu.async_copy(x_h.at[pl.ds(base, BLK)], px_b.at[sl], sem)
            pltpu.async_copy(y_h.at[pl.ds(base, BLK)], py_b.at[sl], sem)
            pltpu.async_copy(z_h.at[pl.ds(base, BLK)], pz_b.at[sl], sem)

        def in_wait(sl):
            sem = isem[sl]
            pltpu.make_async_copy(idx_h.at[pl.ds(0, BLK)], idx_b.at[sl], sem).wait()
            pltpu.make_async_copy(x_h.at[pl.ds(0, BLK)], px_b.at[sl], sem).wait()
            pltpu.make_async_copy(y_h.at[pl.ds(0, BLK)], py_b.at[sl], sem).wait()
            pltpu.make_async_copy(z_h.at[pl.ds(0, BLK)], pz_b.at[sl], sem).wait()

        def out_start(k, sl):
            base = hbase(k)
            for buf, hb in zip(out_bufs, out_hs):
                pltpu.async_copy(buf.at[sl], hb.at[pl.ds(base, BLK)], osem[sl])

        def out_wait(sl):
            for buf, hb in zip(out_bufs, out_hs):
                pltpu.make_async_copy(buf.at[sl], hb.at[pl.ds(0, BLK)], osem[sl]).wait()

        def decode(sl, i):
            s = i * 16
            idx = idx_b.at[sl][pl.ds(s, 16)]
            iz = lax.bitwise_and(idx, 127)
            iy = lax.bitwise_and(lax.shift_right_logical(idx, 7), 127)
            ix = lax.shift_right_logical(idx, 14)
            a = plsc.load_gather(xl_v, [ix])
            b = plsc.load_gather(yl_v, [iy])
            c = plsc.load_gather(zl_v, [iz])
            px = px_b.at[sl][pl.ds(s, 16)] + ix.astype(_F32)
            py = py_b.at[sl][pl.ds(s, 16)] + iy.astype(_F32)
            pz = pz_b.at[sl][pl.ds(s, 16)] + iz.astype(_F32)
            return a, b, c, px, py, pz

        def compute(sl):
            if is_sdf:
                @plsc.parallel_loop(0, CHUNKS, 1, unroll=4)
                def _(i):
                    a, b, c, px, py, pz = decode(sl, i)
                    val = (px * (a * px + d) + py * (b * py + e)
                           + pz * (c * pz + f) + g)
                    out_bufs[0].at[sl][pl.ds(i * 16, 16)] = val
            else:
                @plsc.parallel_loop(0, CHUNKS, 1, unroll=4)
                def _(i):
                    a, b, c, px, py, pz = decode(sl, i)
                    gx = 2.0 * a * px + d
                    gy = 2.0 * b * py + e
                    gz = 2.0 * c * pz + f
                    s2 = gx * gx + gy * gy + gz * gz
                    norm = s2 * _rsqrt(s2)
                    inv = 1.0 / (norm + 1e-8)
                    s = i * 16
                    out_bufs[0].at[sl][pl.ds(s, 16)] = gx * inv
                    out_bufs[1].at[sl][pl.ds(s, 16)] = gy * inv
                    out_bufs[2].at[sl][pl.ds(s, 16)] = gz * inv

        @pl.when(nblk_w > 0)
        def _():
            in_start(0, 0)

        def pair(k2, carry):
            for b2 in (0, 1):
                k = 2 * k2 + b2

                @pl.when(k + 1 < nblk_w)
                def _():
                    in_start(k + 1, 1 - b2)

                @pl.when(k < nblk_w)
                def _():
                    in_wait(b2)

                    @pl.when(k >= 2)
                    def _():
                        out_wait(b2)

                    compute(b2)
                    out_start(k, b2)
            return carry

        lax.fori_loop(0, (NT + 1) // 2, pair, 0)
        out_wait(0)

        @pl.when(nblk_w >= 2)
        def _():
            out_wait(1)

    return _body


def _make_call(is_sdf):
    n_out = 1 if is_sdf else 3
    return functools.partial(
        pl.kernel,
        out_type=[jax.ShapeDtypeStruct((NPTS,), _F32)] * n_out,
        mesh=plsc.VectorSubcoreMesh(core_axis_name="c", subcore_axis_name="s"),
        compiler_params=pltpu.CompilerParams(
            needs_layout_passes=False, use_tc_tiling_on_sc=False),
        scratch_types=[
            pltpu.VMEM((RESO,), _F32),
            pltpu.VMEM((RESO,), _F32),
            pltpu.VMEM((RESO,), _F32),
            pltpu.VMEM((4, 16), _F32),
            pltpu.VMEM((2, BLK), _I32),
            pltpu.VMEM((2, BLK), _F32),
            pltpu.VMEM((2, BLK), _F32),
            pltpu.VMEM((2, BLK), _F32),
        ] + [pltpu.VMEM((2, BLK), _F32)] * n_out
        + [
            pltpu.SemaphoreType.DMA,
            pltpu.SemaphoreType.DMA,
            pltpu.SemaphoreType.DMA,
            pltpu.SemaphoreType.DMA,
        ],
    )(_make_body(is_sdf))


_sdf_call = _make_call(True)
_nrm_call = _make_call(False)


def kernel(renderPointList, renderIndexList, sdfPointList, sdfIndexList,
           xLayer, yLayer, zLayer, offset):
    off16 = jnp.broadcast_to(offset[:, None], (4, 16))
    sdf, = _sdf_call(
        sdfPointList[:, 0], sdfPointList[:, 1], sdfPointList[:, 2],
        sdfIndexList, xLayer, yLayer, zLayer, off16)
    nx, ny, nz = _nrm_call(
        renderPointList[:, 0], renderPointList[:, 1], renderPointList[:, 2],
        renderIndexList, xLayer, yLayer, zLayer, off16)
    ci = lax.broadcasted_iota(jnp.int32, (NPTS, 3), 1)
    nrm = jnp.where(ci == 0, nx[:, None],
                    jnp.where(ci == 1, ny[:, None], nz[:, None]))
    return (sdf, nrm)


# speedup vs baseline: 96.7626x; 1.7159x over previous
import functools

import jax
import jax.numpy as jnp
from jax import lax
from jax.experimental import pallas as pl
from jax.experimental.pallas import tpu as pltpu
from jax.experimental.pallas import tpu_sc as plsc

RESO = 128
NPTS = 2_000_000
BLK = 3200
NTILE = NPTS // 128
QB = BLK // 128
NBLK = NPTS // BLK
NWORKERS = 32
NT = (NBLK + NWORKERS - 1) // NWORKERS
CHUNKS = BLK // 16

_F32 = jnp.float32
_I32 = jnp.int32


def _rsqrt(s):
    i = lax.bitcast_convert_type(s, _I32)
    i = jnp.int32(0x5F3759DF) - lax.shift_right_arithmetic(i, 1)
    y = lax.bitcast_convert_type(i, _F32)
    for _ in range(3):
        y = y * (1.5 - 0.5 * s * y * y)
    return y


def _make_body(is_sdf):
    n_out = 1 if is_sdf else 3

    def _body(*args):
        (pts_h, idx_h, xl_h, yl_h, zl_h, off_h), rest = args[:6], args[6:]
        out_hs, rest = rest[:n_out], rest[n_out:]
        (xl_v, yl_v, zl_v, off_v) = rest[:4]
        idx_b = rest[4:6]
        pts_b = rest[6:8]
        rest = rest[8:]
        out_bufs = tuple(zip(rest[:n_out], rest[n_out:2 * n_out]))
        isem0, isem1, osem0, osem1 = rest[2 * n_out:]

        w = lax.axis_index("s") * 2 + lax.axis_index("c")
        pltpu.sync_copy(xl_h, xl_v)
        pltpu.sync_copy(yl_h, yl_v)
        pltpu.sync_copy(zl_h, zl_v)
        pltpu.sync_copy(off_h, off_v)

        isem = (isem0, isem1)
        osem = (osem0, osem1)
        d = off_v[0]
        e = off_v[1]
        f = off_v[2]
        g = off_v[3]

        nblk_w = (NBLK - 1 - w) // NWORKERS + 1

        def hbase(k):
            return (w + k * NWORKERS) * BLK

        def in_start(k, sl):
            base = hbase(k)
            sem = isem[sl]
            pltpu.async_copy(idx_h.at[pl.ds(base, BLK)], idx_b[sl], sem)
            pltpu.async_copy(pts_h.at[pl.ds(3 * base, 3 * BLK)], pts_b[sl], sem)

        def in_wait(sl):
            sem = isem[sl]
            pltpu.make_async_copy(idx_h.at[pl.ds(0, BLK)], idx_b[sl], sem).wait()
            pltpu.make_async_copy(pts_h.at[pl.ds(0, 3 * BLK)], pts_b[sl],
                                  sem).wait()

        def out_start(k, sl):
            base = hbase(k)
            for bufs, hb in zip(out_bufs, out_hs):
                pltpu.async_copy(bufs[sl], hb.at[pl.ds(base, BLK)], osem[sl])

        def out_wait(sl):
            for bufs, hb in zip(out_bufs, out_hs):
                pltpu.make_async_copy(bufs[sl], hb.at[pl.ds(0, BLK)], osem[sl]).wait()

        def decode(sl, i):
            s = i * 16
            off = (lax.shift_right_logical(i, 3) * 384
                   + lax.bitwise_and(i, 7) * 16)
            idx = idx_b[sl][pl.ds(s, 16)]
            iz = lax.bitwise_and(idx, 127)
            iy = lax.bitwise_and(lax.shift_right_logical(idx, 7), 127)
            ix = lax.shift_right_logical(idx, 14)
            a = plsc.load_gather(xl_v, [ix])
            b = plsc.load_gather(yl_v, [iy])
            c = plsc.load_gather(zl_v, [iz])
            px = pts_b[sl][pl.ds(off, 16)] + ix.astype(_F32)
            py = pts_b[sl][pl.ds(off + 128, 16)] + iy.astype(_F32)
            pz = pts_b[sl][pl.ds(off + 256, 16)] + iz.astype(_F32)
            return a, b, c, px, py, pz

        def compute(sl):
            if is_sdf:
                @plsc.parallel_loop(0, CHUNKS, 1, unroll=4)
                def _(i):
                    a, b, c, px, py, pz = decode(sl, i)
                    val = (px * (a * px + d) + py * (b * py + e)
                           + pz * (c * pz + f) + g)
                    out_bufs[0][sl][pl.ds(i * 16, 16)] = val
            else:
                @plsc.parallel_loop(0, CHUNKS, 1, unroll=4)
                def _(i):
                    a, b, c, px, py, pz = decode(sl, i)
                    gx = 2.0 * a * px + d
                    gy = 2.0 * b * py + e
                    gz = 2.0 * c * pz + f
                    s2 = gx * gx + gy * gy + gz * gz
                    norm = s2 * _rsqrt(s2)
                    inv = 1.0 / (norm + 1e-8)
                    s = i * 16
                    out_bufs[0][sl][pl.ds(s, 16)] = gx * inv
                    out_bufs[1][sl][pl.ds(s, 16)] = gy * inv
                    out_bufs[2][sl][pl.ds(s, 16)] = gz * inv

        @pl.when(nblk_w > 0)
        def _():
            in_start(0, 0)

        def pair(k2, carry):
            for b2 in (0, 1):
                k = 2 * k2 + b2

                @pl.when(k + 1 < nblk_w)
                def _():
                    in_start(k + 1, 1 - b2)

                @pl.when(k < nblk_w)
                def _():
                    in_wait(b2)

                    @pl.when(k >= 2)
                    def _():
                        out_wait(b2)

                    compute(b2)
                    out_start(k, b2)
            return carry

        lax.fori_loop(0, (NT + 1) // 2, pair, 0)
        out_wait(0)

        @pl.when(nblk_w >= 2)
        def _():
            out_wait(1)

    return _body


def _make_call(is_sdf):
    n_out = 1 if is_sdf else 3
    return functools.partial(
        pl.kernel,
        out_type=[jax.ShapeDtypeStruct((NPTS,), _F32)] * n_out,
        mesh=plsc.VectorSubcoreMesh(core_axis_name="c", subcore_axis_name="s"),
        compiler_params=pltpu.CompilerParams(
            needs_layout_passes=False, use_tc_tiling_on_sc=False),
        scratch_types=[
            pltpu.VMEM((RESO,), _F32),
            pltpu.VMEM((RESO,), _F32),
            pltpu.VMEM((RESO,), _F32),
            pltpu.VMEM((4, 16), _F32),
        ] + [pltpu.VMEM((BLK,), _I32)] * 2
        + [pltpu.VMEM((3 * BLK,), _F32)] * 2
        + [pltpu.VMEM((BLK,), _F32)] * (2 * n_out)
        + [
            pltpu.SemaphoreType.DMA,
            pltpu.SemaphoreType.DMA,
            pltpu.SemaphoreType.DMA,
            pltpu.SemaphoreType.DMA,
        ],
    )(_make_body(is_sdf))


_sdf_call = _make_call(True)
_nrm_call = _make_call(False)


def kernel(renderPointList, renderIndexList, sdfPointList, sdfIndexList,
           xLayer, yLayer, zLayer, offset):
    off16 = jnp.broadcast_to(offset[:, None], (4, 16))
    spts = sdfPointList.reshape(NTILE, 128, 3).transpose(0, 2, 1).reshape(-1)
    rpts = renderPointList.reshape(NTILE, 128, 3).transpose(0, 2, 1).reshape(-1)
    sdf, = _sdf_call(
        spts, sdfIndexList, xLayer, yLayer, zLayer, off16)
    nx, ny, nz = _nrm_call(
        rpts, renderIndexList, xLayer, yLayer, zLayer, off16)
    ci = lax.broadcasted_iota(jnp.int32, (NPTS, 3), 1)
    nrm = jnp.where(ci == 0, nx[:, None],
                    jnp.where(ci == 1, ny[:, None], nz[:, None]))
    return (sdf, nrm)
